# baseline (speedup 1.0000x reference)
import functools

import jax
import jax.numpy as jnp
from jax import lax
from jax.experimental import pallas as pl
from jax.experimental.pallas import tpu as pltpu
from jax.experimental.pallas import tpu_sc as plsc

B = 16384
D = 64
NC = 2
NS = 16
L = 16
NW = NC * NS
BPW = B // NW
GCH = 128
NG = BPW // GCH


@functools.partial(
    pl.kernel,
    out_type=jax.ShapeDtypeStruct((B, D), jnp.float32),
    mesh=plsc.VectorSubcoreMesh(core_axis_name="c", subcore_axis_name="s",
                                num_cores=NC, num_subcores=NS),
    scratch_types=[
        pltpu.VMEM((NG, GCH), jnp.int32),
        pltpu.VMEM((BPW, D), jnp.float32),
        pltpu.VMEM((BPW, D), jnp.float32),
        pltpu.SemaphoreType.DMA,
        pltpu.SemaphoreType.DMA,
    ],
    compiler_params=pltpu.CompilerParams(use_tc_tiling_on_sc=False),
)
def _emb_mul(idx_hbm, lat_hbm, tab_hbm, out_hbm, idx_v, rows_v, lat_v, gsem, lsem):
    wid = lax.axis_index("s") * NC + lax.axis_index("c")
    base = wid * BPW

    pltpu.sync_copy(idx_hbm.at[wid], idx_v)
    lat_cp = pltpu.async_copy(lat_hbm.at[pl.ds(base, BPW)], lat_v, lsem)
    gathers = [
        pltpu.async_copy(tab_hbm.at[idx_v.at[g]],
                         rows_v.at[pl.ds(g * GCH, GCH)], gsem)
        for g in range(NG)
    ]
    lat_cp.wait()
    for cp in gathers:
        cp.wait()

    def row(i, carry):
        for j in range(D // L):
            s = pl.ds(j * L, L)
            rows_v[i, s] = rows_v[i, s] * lat_v[i, s]
        return carry

    lax.fori_loop(0, BPW, row, 0)

    pltpu.sync_copy(rows_v, out_hbm.at[pl.ds(base, BPW)])


def kernel(label, latent, emb_table):
    idx = label.astype(jnp.int32).reshape(NW, NG, GCH)
    return _emb_mul(idx, latent, emb_table)
